# baseline (speedup 1.0000x reference)
import functools

import numpy as np
import jax
import jax.numpy as jnp
from jax import lax
from jax.experimental import pallas as pl
from jax.experimental.pallas import tpu as pltpu
from jax.experimental.pallas import tpu_sc as plsc

D_MODEL = 768
MAX_LEN = 5000
BATCH = 4
SEQ = 4096
ROWS = BATCH * SEQ
XSCALE = float(np.sqrt(float(D_MODEL)))

NC = 2
NS = 16
NW = NC * NS
RPW = ROWS // NW
CH = 16
NCHUNK = RPW // CH
NSLOT = 4
NQUAD = NCHUNK // NSLOT
LANES = 16
DV = D_MODEL // LANES


def _pe_table_np():
    position = np.arange(MAX_LEN, dtype=np.float32)[:, None]
    div_term = np.exp(
        np.arange(0, D_MODEL, 2, dtype=np.float32) * (-np.log(10000.0) / D_MODEL)
    )
    pe = np.zeros((MAX_LEN, D_MODEL), dtype=np.float32)
    pe[:, 0::2] = np.sin(position * div_term)
    pe[:, 1::2] = np.cos(position * div_term)
    return pe


_PE_NP = _pe_table_np()


@functools.partial(
    pl.kernel,
    mesh=plsc.VectorSubcoreMesh(core_axis_name="c", subcore_axis_name="s"),
    out_type=jax.ShapeDtypeStruct((ROWS, D_MODEL), jnp.float32),
    scratch_types=(
        [pltpu.VMEM((RPW,), jnp.int32)]
        + [pltpu.VMEM((CH, D_MODEL), jnp.float32) for _ in range(2 * NSLOT)]
        + [pltpu.SemaphoreType.DMA for _ in range(2 * NSLOT)]
    ),
)
def _sc_encode(x_hbm, idx_hbm, pe_hbm, out_hbm, idx_v, *bufs):
    x_v = bufs[0:NSLOT]
    pe_v = bufs[NSLOT:2 * NSLOT]
    lsem = bufs[2 * NSLOT:3 * NSLOT]
    ssem = bufs[3 * NSLOT:4 * NSLOT]

    cid = lax.axis_index("c")
    sid = lax.axis_index("s")
    wid = sid * NC + cid
    base = wid * RPW

    pltpu.sync_copy(idx_hbm.at[pl.ds(base, RPW)], idx_v)

    def issue_loads(c, k):
        pltpu.async_copy(pe_hbm.at[idx_v.at[pl.ds(c * CH, CH)]], pe_v[k], lsem[k])
        pltpu.async_copy(x_hbm.at[pl.ds(base + c * CH, CH)], x_v[k], lsem[k])

    def wait_loads(c, k):
        pltpu.make_async_copy(
            pe_hbm.at[idx_v.at[pl.ds(c * CH, CH)]], pe_v[k], lsem[k]).wait()
        pltpu.make_async_copy(
            x_hbm.at[pl.ds(base + c * CH, CH)], x_v[k], lsem[k]).wait()

    def wait_store(c, k):
        pltpu.make_async_copy(
            pe_v[k], out_hbm.at[pl.ds(base + c * CH, CH)], ssem[k]).wait()

    def compute(k):
        def row_body(r, rcarry):
            for j in range(DV):
                sl = pl.ds(j * LANES, LANES)
                plsc.addupdate(pe_v[k].at[r, sl], x_v[k][r, sl] * XSCALE)
            return rcarry

        lax.fori_loop(0, CH, row_body, 0)

    issue_loads(0, 0)
    issue_loads(1, 1)

    def quad_body(q, carry):
        for k in range(NSLOT):
            c = q * NSLOT + k
            kp = (k + 2) % NSLOT

            @pl.when(c + 2 < NCHUNK)
            def _(c=c, kp=kp):
                @pl.when(c >= 2)
                def _():
                    wait_store(c - 2, kp)

                issue_loads(c + 2, kp)

            wait_loads(c, k)
            compute(k)
            pltpu.async_copy(pe_v[k], out_hbm.at[pl.ds(base + c * CH, CH)],
                             ssem[k])
        return carry

    lax.fori_loop(0, NQUAD, quad_body, 0)

    wait_store(NCHUNK - 2, (NCHUNK - 2) % NSLOT)
    wait_store(NCHUNK - 1, (NCHUNK - 1) % NSLOT)


def kernel(x, index):
    pe = jnp.asarray(_PE_NP)
    xf = x.reshape(ROWS, D_MODEL)
    idxf = index.reshape(ROWS).astype(jnp.int32)
    out = _sc_encode(xf, idxf, pe)
    return out.reshape(x.shape)
